# baseline (speedup 1.0000x reference)
import functools
import math

import jax
import jax.numpy as jnp
from jax import lax
from jax.experimental import pallas as pl
from jax.experimental.pallas import tpu as pltpu
from jax.experimental.pallas import tpu_sc as plsc

B = 32
N = 8400
C = 80
D = 85
DP = 128
M = 200
SEC = 208
STRIDES = (8, 16, 32)
GRIDS = (80, 40, 20)
OFFSETS = (0, 6400, 8000)
EPS = 1e-7
NC = 2
NS = 16
NW = NC * NS
SLOTS_PER_W = 32
SLOTS = 640


def _softplus(v):
    return jnp.maximum(v, 0.0) + jnp.log1p(jnp.exp(-jnp.abs(v)))


def _objd_body(pl4_ref, dsum_ref):
    dsum_ref[0, 0] = jnp.sum(_softplus(pl4_ref[0]))


_obj_dense = pl.pallas_call(
    _objd_body,
    grid=(1,),
    in_specs=[pl.BlockSpec((1, B, N), lambda i: (4, 0, 0))],
    out_specs=pl.BlockSpec(memory_space=pltpu.SMEM),
    out_shape=jax.ShapeDtypeStruct((1, 1), jnp.float32),
)


@functools.partial(
    pl.kernel,
    out_type=(
        jax.ShapeDtypeStruct((NW * SLOTS_PER_W,), jnp.int32),
        jax.ShapeDtypeStruct((NW * SLOTS_PER_W,), jnp.int32),
    ),
    mesh=plsc.VectorSubcoreMesh(core_axis_name="c", subcore_axis_name="s"),
    compiler_params=pltpu.CompilerParams(use_tc_tiling_on_sc=True),
    scratch_types=[
        pltpu.VMEM((3 * NW * SLOTS_PER_W,), jnp.float32),
        pltpu.VMEM((SLOTS_PER_W,), jnp.int32),
        pltpu.VMEM((SLOTS_PER_W,), jnp.int32),
        pltpu.VMEM((16,), jnp.float32),
    ],
)
def _sc_idx(tcols_hbm, isz_hbm, bv_hbm, nv_hbm, tv, bv, nv, isz_v):
    wid = lax.axis_index("s") * NC + lax.axis_index("c")
    pltpu.sync_copy(tcols_hbm, tv)
    pltpu.sync_copy(isz_hbm, isz_v)
    isz = isz_v[...]
    lanes = lax.iota(jnp.int32, 16)
    npd = NW * SLOTS_PER_W

    for j2 in range(SLOTS_PER_W // 16):
        base_k = wid * SLOTS_PER_W + j2 * 16
        k = base_k + lanes
        s_id = jnp.where(k >= SEC, 1, 0) + jnp.where(k >= 2 * SEC, 1, 0)
        tb = tv[pl.ds(base_k, 16)]
        tx = tv[pl.ds(npd + base_k, 16)]
        ty = tv[pl.ds(2 * npd + base_k, 16)]
        sf = jnp.where(s_id == 0, jnp.float32(8.0),
                       jnp.where(s_id == 1, jnp.float32(16.0),
                                 jnp.float32(32.0)))
        g = jnp.where(s_id == 0, 80, jnp.where(s_id == 1, 40, 20))
        off = jnp.where(s_id == 0, 0, jnp.where(s_id == 1, 6400, 8000))
        gx = jnp.clip(((tx * isz) / sf).astype(jnp.int32), 0, g - 1)
        gy = jnp.clip(((ty * isz) / sf).astype(jnp.int32), 0, g - 1)
        bv[pl.ds(j2 * 16, 16)] = tb.astype(jnp.int32)
        nv[pl.ds(j2 * 16, 16)] = gy * g + gx + off

    pltpu.sync_copy(bv, bv_hbm.at[pl.ds(wid * SLOTS_PER_W, SLOTS_PER_W)])
    pltpu.sync_copy(nv, nv_hbm.at[pl.ds(wid * SLOTS_PER_W, SLOTS_PER_W)])


GB = 16
GSTEPS = SLOTS // GB


def _gather_body(bv_ref, nv_ref, *refs):
    blks = refs[:GB]
    out_ref = refs[GB]
    k = pl.program_id(0)
    lane_iota = lax.broadcasted_iota(jnp.int32, (1, 128), 1)
    for i in range(GB):
        s = k * GB + i
        b7 = bv_ref[s] & 7
        n_loc = nv_ref[s] & 127
        m = blks[i][:, pl.ds(b7, 1), :][:, 0, :]
        e = jnp.where(lane_iota == n_loc, 1.0, 0.0)
        sel = jax.lax.dot_general(e, m, (((1,), (1,)), ((), ())),
                                  preferred_element_type=jnp.float32)
        out_ref[0, i:i + 1, 0:D] = sel


def _mk_gspec(i):
    return pl.BlockSpec(
        (D, 8, 128),
        lambda k, bv, nv, i=i: (0, bv[k * GB + i] >> 3, nv[k * GB + i] >> 7),
    )


_tc_gather = pl.pallas_call(
    _gather_body,
    grid_spec=pltpu.PrefetchScalarGridSpec(
        num_scalar_prefetch=2,
        grid=(GSTEPS,),
        in_specs=[_mk_gspec(i) for i in range(GB)],
        out_specs=pl.BlockSpec((1, GB, DP), lambda k, bv, nv: (k, 0, 0)),
    ),
    out_shape=jax.ShapeDtypeStruct((GSTEPS, GB, DP), jnp.float32),
)


def _atan_pos(x):
    t3 = 2.414213562373095
    t1 = 0.4142135623730950
    c = jnp.where(x > t3, -1.0 / x, jnp.where(x > t1, (x - 1.0) / (x + 1.0),
                                              x))
    base = jnp.where(x > t3, math.pi / 2,
                     jnp.where(x > t1, math.pi / 4, 0.0))
    zz = c * c
    p = (((8.05374449538e-2 * zz - 1.38776856032e-1) * zz
          + 1.99777106478e-1) * zz - 3.33329491539e-1) * zz * c + c
    return base + p


def _fin_body(rows_ref, tgt_ref, dsum_ref, isz_ref, out_ref):
    isz = isz_ref[0, 0]
    dsum = dsum_ref[0, 0]
    t = tgt_ref[...]
    bcol = t[:, 0].astype(jnp.int32)
    ccol = t[:, 1].astype(jnp.int32)
    tx, ty, tw, th = t[:, 2], t[:, 3], t[:, 4], t[:, 5]

    mi = lax.broadcasted_iota(jnp.int32, (SEC, SEC), 0)
    mj = lax.broadcasted_iota(jnp.int32, (SEC, SEC), 1)
    validj = (mj < M) & (mj > mi)
    valid_f = (lax.iota(jnp.int32, SEC) < M).astype(jnp.float32)
    eq_c = ccol[:, None] == ccol[None, :]
    cls_iota = lax.broadcasted_iota(jnp.int32, (SEC, C), 1)
    onehot = jnp.where(ccol[:, None] == cls_iota, 1.0, 0.0)

    num_pos = jnp.float32(0.0)
    s_obj = jnp.float32(0.0)
    s_box = jnp.float32(0.0)
    s_sp = jnp.float32(0.0)
    s_lg = jnp.float32(0.0)
    for si in range(3):
        s = float(STRIDES[si])
        g = GRIDS[si]
        off = OFFSETS[si]
        gx = jnp.clip(((tx * isz) / jnp.float32(s)).astype(jnp.int32), 0,
                      g - 1)
        gy = jnp.clip(((ty * isz) / jnp.float32(s)).astype(jnp.int32), 0,
                      g - 1)
        idx = bcol * N + gy * g + gx + off
        eq = idx[:, None] == idx[None, :]
        dup = jnp.where(eq & validj, 1.0, 0.0)
        win_f = valid_f * jnp.where(jnp.sum(dup, axis=1) > 0.0, 0.0, 1.0)
        dupc = jnp.where(eq & eq_c & validj, 1.0, 0.0)
        keep_f = valid_f * jnp.where(jnp.sum(dupc, axis=1) > 0.0, 0.0, 1.0)

        sl = slice(si * SEC, (si + 1) * SEC)
        lx, ly = rows_ref[sl, 0], rows_ref[sl, 1]
        lw, lh = rows_ref[sl, 2], rows_ref[sl, 3]
        obj = rows_ref[sl, 4]
        cls = rows_ref[sl, 5:5 + C]

        px = (1.0 / (1.0 + jnp.exp(-lx)) + gx.astype(jnp.float32)) * s / isz
        py = (1.0 / (1.0 + jnp.exp(-ly)) + gy.astype(jnp.float32)) * s / isz
        pw = 1.0 / (1.0 + jnp.exp(-lw))
        ph = 1.0 / (1.0 + jnp.exp(-lh))

        px1, px2 = px - pw / 2, px + pw / 2
        py1, py2 = py - ph / 2, py + ph / 2
        tx1, tx2 = tx - tw / 2, tx + tw / 2
        ty1, ty2 = ty - th / 2, ty + th / 2
        ix1 = jnp.maximum(px1, tx1)
        iy1 = jnp.maximum(py1, ty1)
        ix2 = jnp.minimum(px2, tx2)
        iy2 = jnp.minimum(py2, ty2)
        inter = jnp.clip(ix2 - ix1, 0.0) * jnp.clip(iy2 - iy1, 0.0)
        pa = (px2 - px1) * (py2 - py1)
        ta = (tx2 - tx1) * (ty2 - ty1)
        union = pa + ta - inter + EPS
        iou = inter / union
        ex1 = jnp.minimum(px1, tx1)
        ey1 = jnp.minimum(py1, ty1)
        ex2 = jnp.maximum(px2, tx2)
        ey2 = jnp.maximum(py2, ty2)
        c2 = (ex2 - ex1) ** 2 + (ey2 - ey1) ** 2 + EPS
        pcx, pcy = (px1 + px2) / 2, (py1 + py2) / 2
        tcx, tcy = (tx1 + tx2) / 2, (ty1 + ty2) / 2
        rho2 = (pcx - tcx) ** 2 + (pcy - tcy) ** 2
        v = (4.0 / math.pi ** 2) * (_atan_pos(tw / (th + EPS)) -
                                    _atan_pos(pw / (ph + EPS))) ** 2
        alpha = v / (1.0 - iou + v + EPS)
        ciou = iou - rho2 / c2 - alpha * v

        num_pos += jnp.sum(win_f)
        s_obj += jnp.sum(win_f * obj)
        s_box += jnp.sum(win_f * (1.0 - ciou))
        s_sp += jnp.sum(win_f[:, None] * _softplus(cls))
        s_lg += jnp.sum(keep_f[:, None] * onehot * cls)

    obj_loss = (dsum - s_obj) / jnp.float32(B * N)
    box_loss = s_box / jnp.maximum(num_pos, 1.0)
    cls_loss = (s_sp - s_lg) / jnp.maximum(num_pos * C, 1.0)
    out_ref[0, 0] = 5.0 * box_loss + obj_loss + cls_loss


_finalize = pl.pallas_call(
    _fin_body,
    in_specs=[
        pl.BlockSpec((SLOTS, DP), lambda: (0, 0)),
        pl.BlockSpec((SEC, 8), lambda: (0, 0)),
        pl.BlockSpec(memory_space=pltpu.SMEM),
        pl.BlockSpec(memory_space=pltpu.SMEM),
    ],
    out_specs=pl.BlockSpec(memory_space=pltpu.SMEM),
    out_shape=jax.ShapeDtypeStruct((1, 1), jnp.float32),
)


def kernel(predictions, targets, input_size):
    iszf = jnp.asarray(input_size, jnp.float32)
    isz16 = jnp.full((16,), iszf, jnp.float32)
    isz11 = jnp.reshape(iszf, (1, 1))
    m_ids = jnp.minimum(jnp.arange(SEC), M - 1)
    slot_m = jnp.concatenate(
        [m_ids, m_ids, m_ids, jnp.zeros(NW * SLOTS_PER_W - 3 * SEC,
                                        jnp.int32)])
    tcols = jnp.concatenate([targets[:, 0][slot_m], targets[:, 2][slot_m],
                             targets[:, 3][slot_m]])
    tpad = jnp.zeros((SEC, 8), jnp.float32).at[:M, :6].set(targets)
    predt = jnp.transpose(predictions, (2, 0, 1))
    dsum = _obj_dense(predt)
    bv, nv = _sc_idx(tcols, isz16)
    rows = _tc_gather(bv, nv, *([predt] * GB)).reshape(SLOTS, DP)
    total = _finalize(rows, tpad, dsum, isz11)
    return total.reshape(())


# speedup vs baseline: 1.0069x; 1.0069x over previous
import functools
import math

import jax
import jax.numpy as jnp
from jax import lax
from jax.experimental import pallas as pl
from jax.experimental.pallas import tpu as pltpu
from jax.experimental.pallas import tpu_sc as plsc

B = 32
N = 8400
C = 80
D = 85
DP = 128
M = 200
SEC = 208
STRIDES = (8, 16, 32)
GRIDS = (80, 40, 20)
OFFSETS = (0, 6400, 8000)
EPS = 1e-7
NC = 2
NS = 16
NW = NC * NS
SLOTS_PER_W = 32
SLOTS = 640


def _softplus(v):
    return jnp.maximum(v, 0.0) + jnp.log1p(jnp.exp(-jnp.abs(v)))


def _objd_body(pl4_ref, dsum_ref):
    dsum_ref[0, 0] = jnp.sum(_softplus(pl4_ref[0]))


_obj_dense = pl.pallas_call(
    _objd_body,
    grid=(1,),
    in_specs=[pl.BlockSpec((1, B, N), lambda i: (4, 0, 0))],
    out_specs=pl.BlockSpec(memory_space=pltpu.SMEM),
    out_shape=jax.ShapeDtypeStruct((1, 1), jnp.float32),
)


@functools.partial(
    pl.kernel,
    out_type=(
        jax.ShapeDtypeStruct((NW * SLOTS_PER_W,), jnp.int32),
        jax.ShapeDtypeStruct((NW * SLOTS_PER_W,), jnp.int32),
    ),
    mesh=plsc.VectorSubcoreMesh(core_axis_name="c", subcore_axis_name="s"),
    compiler_params=pltpu.CompilerParams(use_tc_tiling_on_sc=True),
    scratch_types=[
        pltpu.VMEM((3 * NW * SLOTS_PER_W,), jnp.float32),
        pltpu.VMEM((SLOTS_PER_W,), jnp.int32),
        pltpu.VMEM((SLOTS_PER_W,), jnp.int32),
        pltpu.VMEM((16,), jnp.float32),
    ],
)
def _sc_idx(tcols_hbm, isz_hbm, bv_hbm, nv_hbm, tv, bv, nv, isz_v):
    wid = lax.axis_index("s") * NC + lax.axis_index("c")
    pltpu.sync_copy(tcols_hbm, tv)
    pltpu.sync_copy(isz_hbm, isz_v)
    isz = isz_v[...]
    lanes = lax.iota(jnp.int32, 16)
    npd = NW * SLOTS_PER_W

    for j2 in range(SLOTS_PER_W // 16):
        base_k = wid * SLOTS_PER_W + j2 * 16
        k = base_k + lanes
        s_id = jnp.where(k >= SEC, 1, 0) + jnp.where(k >= 2 * SEC, 1, 0)
        tb = tv[pl.ds(base_k, 16)]
        tx = tv[pl.ds(npd + base_k, 16)]
        ty = tv[pl.ds(2 * npd + base_k, 16)]
        sf = jnp.where(s_id == 0, jnp.float32(8.0),
                       jnp.where(s_id == 1, jnp.float32(16.0),
                                 jnp.float32(32.0)))
        g = jnp.where(s_id == 0, 80, jnp.where(s_id == 1, 40, 20))
        off = jnp.where(s_id == 0, 0, jnp.where(s_id == 1, 6400, 8000))
        gx = jnp.clip(((tx * isz) / sf).astype(jnp.int32), 0, g - 1)
        gy = jnp.clip(((ty * isz) / sf).astype(jnp.int32), 0, g - 1)
        bv[pl.ds(j2 * 16, 16)] = tb.astype(jnp.int32)
        nv[pl.ds(j2 * 16, 16)] = gy * g + gx + off

    pltpu.sync_copy(bv, bv_hbm.at[pl.ds(wid * SLOTS_PER_W, SLOTS_PER_W)])
    pltpu.sync_copy(nv, nv_hbm.at[pl.ds(wid * SLOTS_PER_W, SLOTS_PER_W)])


GB = 20
GSTEPS = SLOTS // GB


def _gather_body(bv_ref, nv_ref, *refs):
    blks = refs[:GB]
    out_ref = refs[GB]
    k = pl.program_id(0)
    lane_iota = lax.broadcasted_iota(jnp.int32, (1, 128), 1)
    for i in range(GB):
        s = k * GB + i
        b7 = bv_ref[s] & 7
        n_loc = nv_ref[s] & 127
        m = blks[i][:, pl.ds(b7, 1), :][:, 0, :]
        e = jnp.where(lane_iota == n_loc, 1.0, 0.0)
        sel = jax.lax.dot_general(e, m, (((1,), (1,)), ((), ())),
                                  preferred_element_type=jnp.float32)
        out_ref[0, i:i + 1, 0:D] = sel


def _mk_gspec(i):
    return pl.BlockSpec(
        (D, 8, 128),
        lambda k, bv, nv, i=i: (0, bv[k * GB + i] >> 3, nv[k * GB + i] >> 7),
    )


_tc_gather = pl.pallas_call(
    _gather_body,
    grid_spec=pltpu.PrefetchScalarGridSpec(
        num_scalar_prefetch=2,
        grid=(GSTEPS,),
        in_specs=[_mk_gspec(i) for i in range(GB)],
        out_specs=pl.BlockSpec((1, GB, DP), lambda k, bv, nv: (k, 0, 0)),
    ),
    out_shape=jax.ShapeDtypeStruct((GSTEPS, GB, DP), jnp.float32),
)


def _atan_pos(x):
    t3 = 2.414213562373095
    t1 = 0.4142135623730950
    c = jnp.where(x > t3, -1.0 / x, jnp.where(x > t1, (x - 1.0) / (x + 1.0),
                                              x))
    base = jnp.where(x > t3, math.pi / 2,
                     jnp.where(x > t1, math.pi / 4, 0.0))
    zz = c * c
    p = (((8.05374449538e-2 * zz - 1.38776856032e-1) * zz
          + 1.99777106478e-1) * zz - 3.33329491539e-1) * zz * c + c
    return base + p


def _fin_body(rows_ref, tgt_ref, dsum_ref, isz_ref, out_ref):
    isz = isz_ref[0, 0]
    dsum = dsum_ref[0, 0]
    t = tgt_ref[...]
    bcol = t[:, 0].astype(jnp.int32)
    ccol = t[:, 1].astype(jnp.int32)
    tx, ty, tw, th = t[:, 2], t[:, 3], t[:, 4], t[:, 5]

    mi = lax.broadcasted_iota(jnp.int32, (SEC, SEC), 0)
    mj = lax.broadcasted_iota(jnp.int32, (SEC, SEC), 1)
    validj = (mj < M) & (mj > mi)
    valid_f = (lax.iota(jnp.int32, SEC) < M).astype(jnp.float32)
    eq_c = ccol[:, None] == ccol[None, :]
    cls_iota = lax.broadcasted_iota(jnp.int32, (SEC, C), 1)
    onehot = jnp.where(ccol[:, None] == cls_iota, 1.0, 0.0)

    num_pos = jnp.float32(0.0)
    s_obj = jnp.float32(0.0)
    s_box = jnp.float32(0.0)
    s_sp = jnp.float32(0.0)
    s_lg = jnp.float32(0.0)
    for si in range(3):
        s = float(STRIDES[si])
        g = GRIDS[si]
        off = OFFSETS[si]
        gx = jnp.clip(((tx * isz) / jnp.float32(s)).astype(jnp.int32), 0,
                      g - 1)
        gy = jnp.clip(((ty * isz) / jnp.float32(s)).astype(jnp.int32), 0,
                      g - 1)
        idx = bcol * N + gy * g + gx + off
        eq = idx[:, None] == idx[None, :]
        dup = jnp.where(eq & validj, 1.0, 0.0)
        win_f = valid_f * jnp.where(jnp.sum(dup, axis=1) > 0.0, 0.0, 1.0)
        dupc = jnp.where(eq & eq_c & validj, 1.0, 0.0)
        keep_f = valid_f * jnp.where(jnp.sum(dupc, axis=1) > 0.0, 0.0, 1.0)

        sl = slice(si * SEC, (si + 1) * SEC)
        lx, ly = rows_ref[sl, 0], rows_ref[sl, 1]
        lw, lh = rows_ref[sl, 2], rows_ref[sl, 3]
        obj = rows_ref[sl, 4]
        cls = rows_ref[sl, 5:5 + C]

        px = (1.0 / (1.0 + jnp.exp(-lx)) + gx.astype(jnp.float32)) * s / isz
        py = (1.0 / (1.0 + jnp.exp(-ly)) + gy.astype(jnp.float32)) * s / isz
        pw = 1.0 / (1.0 + jnp.exp(-lw))
        ph = 1.0 / (1.0 + jnp.exp(-lh))

        px1, px2 = px - pw / 2, px + pw / 2
        py1, py2 = py - ph / 2, py + ph / 2
        tx1, tx2 = tx - tw / 2, tx + tw / 2
        ty1, ty2 = ty - th / 2, ty + th / 2
        ix1 = jnp.maximum(px1, tx1)
        iy1 = jnp.maximum(py1, ty1)
        ix2 = jnp.minimum(px2, tx2)
        iy2 = jnp.minimum(py2, ty2)
        inter = jnp.clip(ix2 - ix1, 0.0) * jnp.clip(iy2 - iy1, 0.0)
        pa = (px2 - px1) * (py2 - py1)
        ta = (tx2 - tx1) * (ty2 - ty1)
        union = pa + ta - inter + EPS
        iou = inter / union
        ex1 = jnp.minimum(px1, tx1)
        ey1 = jnp.minimum(py1, ty1)
        ex2 = jnp.maximum(px2, tx2)
        ey2 = jnp.maximum(py2, ty2)
        c2 = (ex2 - ex1) ** 2 + (ey2 - ey1) ** 2 + EPS
        pcx, pcy = (px1 + px2) / 2, (py1 + py2) / 2
        tcx, tcy = (tx1 + tx2) / 2, (ty1 + ty2) / 2
        rho2 = (pcx - tcx) ** 2 + (pcy - tcy) ** 2
        v = (4.0 / math.pi ** 2) * (_atan_pos(tw / (th + EPS)) -
                                    _atan_pos(pw / (ph + EPS))) ** 2
        alpha = v / (1.0 - iou + v + EPS)
        ciou = iou - rho2 / c2 - alpha * v

        num_pos += jnp.sum(win_f)
        s_obj += jnp.sum(win_f * obj)
        s_box += jnp.sum(win_f * (1.0 - ciou))
        s_sp += jnp.sum(win_f[:, None] * _softplus(cls))
        s_lg += jnp.sum(keep_f[:, None] * onehot * cls)

    obj_loss = (dsum - s_obj) / jnp.float32(B * N)
    box_loss = s_box / jnp.maximum(num_pos, 1.0)
    cls_loss = (s_sp - s_lg) / jnp.maximum(num_pos * C, 1.0)
    out_ref[0, 0] = 5.0 * box_loss + obj_loss + cls_loss


_finalize = pl.pallas_call(
    _fin_body,
    in_specs=[
        pl.BlockSpec((SLOTS, DP), lambda: (0, 0)),
        pl.BlockSpec((SEC, 8), lambda: (0, 0)),
        pl.BlockSpec(memory_space=pltpu.SMEM),
        pl.BlockSpec(memory_space=pltpu.SMEM),
    ],
    out_specs=pl.BlockSpec(memory_space=pltpu.SMEM),
    out_shape=jax.ShapeDtypeStruct((1, 1), jnp.float32),
)


def kernel(predictions, targets, input_size):
    iszf = jnp.asarray(input_size, jnp.float32)
    isz16 = jnp.full((16,), iszf, jnp.float32)
    isz11 = jnp.reshape(iszf, (1, 1))
    m_ids = jnp.minimum(jnp.arange(SEC), M - 1)
    slot_m = jnp.concatenate(
        [m_ids, m_ids, m_ids, jnp.zeros(NW * SLOTS_PER_W - 3 * SEC,
                                        jnp.int32)])
    tcols = jnp.concatenate([targets[:, 0][slot_m], targets[:, 2][slot_m],
                             targets[:, 3][slot_m]])
    tpad = jnp.zeros((SEC, 8), jnp.float32).at[:M, :6].set(targets)
    predt = jnp.transpose(predictions, (2, 0, 1))
    dsum = _obj_dense(predt)
    bv, nv = _sc_idx(tcols, isz16)
    rows = _tc_gather(bv, nv, *([predt] * GB)).reshape(SLOTS, DP)
    total = _finalize(rows, tpad, dsum, isz11)
    return total.reshape(())
